# baseline (speedup 1.0000x reference)
import functools

import jax
import jax.numpy as jnp
from jax import lax
from jax.experimental import pallas as pl
from jax.experimental.pallas import tpu as pltpu
from jax.experimental.pallas import tpu_sc as plsc

B, C, H, W = 8, 64, 256, 256
N = 128
NROW = B * C * H
NPAIR = B * N
NW = 32
PAIRS_PER_W = NPAIR // NW
ELEMS_PER_W = PAIRS_PER_W * C
L = 16
WIN = 128
WAVE = 4
NWAVE = PAIRS_PER_W // WAVE


def _splat(x):
    return jnp.broadcast_to(jnp.asarray(x, jnp.int32), (L,))


def _stage1(out_hbm, ind_hbm, targ_hbm, wt_hbm, part_hbm,
            ind_v, wt_v, targ_v, idx_v, col_v, wl_v, win_v, gath_v,
            part_v, sem, semb):
    wid = lax.axis_index("c") * 16 + lax.axis_index("s")
    base = wid * PAIRS_PER_W
    b = base // N

    pltpu.sync_copy(ind_hbm.at[pl.ds(base, PAIRS_PER_W)], ind_v)
    pltpu.sync_copy(wt_hbm.at[pl.ds(base, PAIRS_PER_W)], wt_v)
    pltpu.sync_copy(targ_hbm.at[pl.ds(base, PAIRS_PER_W), pl.ds(0, C)],
                    targ_v)

    ci = lax.iota(jnp.int32, L) * H
    row0 = b * (C * H)
    for blk in range(PAIRS_PER_W // L):
        sv = ind_v[pl.ds(blk * L, L)]
        col_v[pl.ds(blk * L, L)] = lax.bitwise_and(sv, jnp.int32(W - WIN))
        wl_v[pl.ds(blk * L, L)] = lax.bitwise_and(sv, jnp.int32(WIN - 1))

    def idx_body(j, carry):
        sj = plsc.load_gather(ind_v, [_splat(j)])
        hj = lax.shift_right_logical(sj, 8) + row0
        for cb in range(C // L):
            idx_v[j, pl.ds(cb * L, L)] = hj + (ci + cb * L * H)
        return carry

    lax.fori_loop(0, PAIRS_PER_W, idx_body, jnp.int32(0))

    ci16 = lax.iota(jnp.int32, L)

    def fire(wv_i, par, sem_p):
        handles = []
        for p in range(WAVE):
            j = wv_i * WAVE + p
            col0 = pl.multiple_of(plsc.load_gather(col_v, [_splat(j)])[0],
                                  WIN)
            handles.append(pltpu.async_copy(
                out_hbm.at[idx_v.at[j], pl.ds(col0, WIN)],
                win_v.at[par, p], sem_p))
        return handles

    def extract(wv_i, par):
        bi = _splat(par)
        for p in range(WAVE):
            j = wv_i * WAVE + p
            wl = plsc.load_gather(wl_v, [_splat(j)])
            pi = _splat(p)
            for cb in range(C // L):
                g16 = plsc.load_gather(win_v, [bi, pi, ci16 + cb * L, wl])
                gath_v[pl.ds(j * C + cb * L, L)] = g16

    def wave_body(k, carry):
        wa = k * 2
        ha = fire(wa, 0, sem)
        hb = fire(wa + 1, 1, semb)
        for hnd in ha:
            hnd.wait()
        extract(wa, 0)
        for hnd in hb:
            hnd.wait()
        extract(wa + 1, 1)
        return carry

    lax.fori_loop(0, NWAVE // 2, wave_body, jnp.int32(0))

    def loss_body(j, acc):
        w16 = plsc.load_gather(wt_v, [_splat(j)])
        for cb in range(C // L):
            g = gath_v[pl.ds(j * C + cb * L, L)]
            t = targ_v[j, pl.ds(cb * L, L)]
            acc = acc + jnp.abs(g * w16 - t * w16)
        return acc

    acc = lax.fori_loop(0, PAIRS_PER_W, loss_body, jnp.zeros((L,), jnp.float32))

    wsum = wt_v[pl.ds(0, L)] + wt_v[pl.ds(L, L)]
    part_v[pl.ds(0, L)] = acc
    part_v[pl.ds(L, L)] = wsum
    pltpu.sync_copy(part_v, part_hbm.at[wid])


def _lane_sum(x, red_v):
    red_v[pl.ds(0, L)] = x
    red_v[pl.ds(L, L)] = jnp.zeros((L,), jnp.float32)
    for s in (8, 4, 2, 1):
        y = red_v[pl.ds(0, L)] + red_v[pl.ds(s, L)]
        red_v[pl.ds(0, L)] = y
    return red_v[pl.ds(0, L)]


def _stage2(part_hbm, out_hbm, pa_v, res_v, red_v):
    wid = lax.axis_index("c") * 16 + lax.axis_index("s")

    @pl.when(wid == 0)
    def _():
        pltpu.sync_copy(part_hbm, pa_v)
        acc = jnp.zeros((L,), jnp.float32)
        wsum = jnp.zeros((L,), jnp.float32)
        for j in range(NW):
            acc = acc + pa_v[j, pl.ds(0, L)]
            wsum = wsum + pa_v[j, pl.ds(L, L)]
        num = _lane_sum(acc, red_v)
        den = _lane_sum(wsum, red_v) * jnp.float32(C) + jnp.float32(0.0001)
        res_v[...] = num / den
        pltpu.sync_copy(res_v, out_hbm)


def kernel(output, target, ind, weight):
    out2d = output.reshape(NROW, W)
    ind32 = ind.astype(jnp.int32).reshape(NPAIR)
    targ = target.astype(jnp.float32).reshape(NPAIR, C)
    wt = weight.astype(jnp.float32).reshape(NPAIR)

    mesh = plsc.VectorSubcoreMesh(core_axis_name="c", subcore_axis_name="s")

    s1 = functools.partial(
        pl.kernel, mesh=mesh,
        out_type=jax.ShapeDtypeStruct((NW, 2 * L), jnp.float32),
        compiler_params=pltpu.CompilerParams(needs_layout_passes=False),
        scratch_types=[
            pltpu.VMEM((PAIRS_PER_W,), jnp.int32),
            pltpu.VMEM((PAIRS_PER_W,), jnp.float32),
            pltpu.VMEM((PAIRS_PER_W, C), jnp.float32),
            pltpu.VMEM((PAIRS_PER_W, C), jnp.int32),
            pltpu.VMEM((PAIRS_PER_W,), jnp.int32),
            pltpu.VMEM((PAIRS_PER_W,), jnp.int32),
            pltpu.VMEM((2, WAVE, C, WIN), jnp.float32),
            pltpu.VMEM((ELEMS_PER_W,), jnp.float32),
            pltpu.VMEM((2 * L,), jnp.float32),
            pltpu.SemaphoreType.DMA,
            pltpu.SemaphoreType.DMA,
        ],
    )(_stage1)
    partials = s1(out2d, ind32, targ, wt)

    s2 = functools.partial(
        pl.kernel, mesh=mesh,
        out_type=jax.ShapeDtypeStruct((L,), jnp.float32),
        scratch_types=[
            pltpu.VMEM((NW, 2 * L), jnp.float32),
            pltpu.VMEM((L,), jnp.float32),
            pltpu.VMEM((2 * L,), jnp.float32),
        ],
    )(_stage2)
    res = s2(partials)
    return res[0]


# speedup vs baseline: 1.0202x; 1.0202x over previous
import functools

import jax
import jax.numpy as jnp
from jax import lax
from jax.experimental import pallas as pl
from jax.experimental.pallas import tpu as pltpu
from jax.experimental.pallas import tpu_sc as plsc

B, C, H, W = 8, 64, 256, 256
N = 128
NROW = B * C * H
NPAIR = B * N
NW = 32
PAIRS_PER_W = NPAIR // NW
ELEMS_PER_W = PAIRS_PER_W * C
L = 16
WIN = 128
WAVE = 4
NWAVE = PAIRS_PER_W // WAVE


def _splat(x):
    return jnp.broadcast_to(jnp.asarray(x, jnp.int32), (L,))


def _stage1(out_hbm, ind_hbm, targ_hbm, wt_hbm, part_hbm,
            ind_v, wt_v, targ_v, idx_v, col_v, wl_v, win_v,
            part_v, sem, semb):
    wid = lax.axis_index("c") * 16 + lax.axis_index("s")
    base = wid * PAIRS_PER_W
    b = base // N

    pltpu.sync_copy(ind_hbm.at[pl.ds(base, PAIRS_PER_W)], ind_v)
    pltpu.sync_copy(wt_hbm.at[pl.ds(base, PAIRS_PER_W)], wt_v)
    pltpu.sync_copy(targ_hbm.at[pl.ds(base, PAIRS_PER_W), pl.ds(0, C)],
                    targ_v)

    ci = lax.iota(jnp.int32, L) * H
    row0 = b * (C * H)
    for blk in range(PAIRS_PER_W // L):
        sv = ind_v[pl.ds(blk * L, L)]
        col_v[pl.ds(blk * L, L)] = lax.bitwise_and(sv, jnp.int32(W - WIN))
        wl_v[pl.ds(blk * L, L)] = lax.bitwise_and(sv, jnp.int32(WIN - 1))

    def idx_body(j, carry):
        sj = plsc.load_gather(ind_v, [_splat(j)])
        hj = lax.shift_right_logical(sj, 8) + row0
        for cb in range(C // L):
            idx_v[j, pl.ds(cb * L, L)] = hj + (ci + cb * L * H)
        return carry

    lax.fori_loop(0, PAIRS_PER_W, idx_body, jnp.int32(0))

    ci16 = lax.iota(jnp.int32, L)

    def fire(wv_i, par, sem_p):
        handles = []
        for p in range(WAVE):
            j = wv_i * WAVE + p
            col0 = pl.multiple_of(plsc.load_gather(col_v, [_splat(j)])[0],
                                  WIN)
            handles.append(pltpu.async_copy(
                out_hbm.at[idx_v.at[j], pl.ds(col0, WIN)],
                win_v.at[par, p], sem_p))
        return handles

    def extract(wv_i, par, acc):
        bi = _splat(par)
        for p in range(WAVE):
            j = wv_i * WAVE + p
            wl = plsc.load_gather(wl_v, [_splat(j)])
            pi = _splat(p)
            w16 = plsc.load_gather(wt_v, [_splat(j)])
            for cb in range(C // L):
                g16 = plsc.load_gather(win_v, [bi, pi, ci16 + cb * L, wl])
                t = targ_v[j, pl.ds(cb * L, L)]
                acc = acc + jnp.abs(g16 * w16 - t * w16)
        return acc

    def wave_body(k, acc):
        wa = k * 2
        ha = fire(wa, 0, sem)
        hb = fire(wa + 1, 1, semb)
        for hnd in ha:
            hnd.wait()
        acc = extract(wa, 0, acc)
        for hnd in hb:
            hnd.wait()
        acc = extract(wa + 1, 1, acc)
        return acc

    acc = lax.fori_loop(0, NWAVE // 2, wave_body,
                        jnp.zeros((L,), jnp.float32))

    wsum = wt_v[pl.ds(0, L)] + wt_v[pl.ds(L, L)]
    part_v[pl.ds(0, L)] = acc
    part_v[pl.ds(L, L)] = wsum
    pltpu.sync_copy(part_v, part_hbm.at[wid])


def _lane_sum(x, red_v):
    red_v[pl.ds(0, L)] = x
    red_v[pl.ds(L, L)] = jnp.zeros((L,), jnp.float32)
    for s in (8, 4, 2, 1):
        y = red_v[pl.ds(0, L)] + red_v[pl.ds(s, L)]
        red_v[pl.ds(0, L)] = y
    return red_v[pl.ds(0, L)]


def _stage2(part_hbm, out_hbm, pa_v, res_v, red_v):
    wid = lax.axis_index("c") * 16 + lax.axis_index("s")

    @pl.when(wid == 0)
    def _():
        pltpu.sync_copy(part_hbm, pa_v)
        acc = jnp.zeros((L,), jnp.float32)
        wsum = jnp.zeros((L,), jnp.float32)
        for j in range(NW):
            acc = acc + pa_v[j, pl.ds(0, L)]
            wsum = wsum + pa_v[j, pl.ds(L, L)]
        num = _lane_sum(acc, red_v)
        den = _lane_sum(wsum, red_v) * jnp.float32(C) + jnp.float32(0.0001)
        res_v[...] = num / den
        pltpu.sync_copy(res_v, out_hbm)


def kernel(output, target, ind, weight):
    out2d = output.reshape(NROW, W)
    ind32 = ind.astype(jnp.int32).reshape(NPAIR)
    targ = target.astype(jnp.float32).reshape(NPAIR, C)
    wt = weight.astype(jnp.float32).reshape(NPAIR)

    mesh = plsc.VectorSubcoreMesh(core_axis_name="c", subcore_axis_name="s")

    s1 = functools.partial(
        pl.kernel, mesh=mesh,
        out_type=jax.ShapeDtypeStruct((NW, 2 * L), jnp.float32),
        compiler_params=pltpu.CompilerParams(needs_layout_passes=False),
        scratch_types=[
            pltpu.VMEM((PAIRS_PER_W,), jnp.int32),
            pltpu.VMEM((PAIRS_PER_W,), jnp.float32),
            pltpu.VMEM((PAIRS_PER_W, C), jnp.float32),
            pltpu.VMEM((PAIRS_PER_W, C), jnp.int32),
            pltpu.VMEM((PAIRS_PER_W,), jnp.int32),
            pltpu.VMEM((PAIRS_PER_W,), jnp.int32),
            pltpu.VMEM((2, WAVE, C, WIN), jnp.float32),
            pltpu.VMEM((2 * L,), jnp.float32),
            pltpu.SemaphoreType.DMA,
            pltpu.SemaphoreType.DMA,
        ],
    )(_stage1)
    partials = s1(out2d, ind32, targ, wt)

    s2 = functools.partial(
        pl.kernel, mesh=mesh,
        out_type=jax.ShapeDtypeStruct((L,), jnp.float32),
        scratch_types=[
            pltpu.VMEM((NW, 2 * L), jnp.float32),
            pltpu.VMEM((L,), jnp.float32),
            pltpu.VMEM((2 * L,), jnp.float32),
        ],
    )(_stage2)
    res = s2(partials)
    return res[0]
